# baseline (speedup 1.0000x reference)
import functools

import jax
import jax.numpy as jnp
from jax import lax
from jax.experimental import pallas as pl
from jax.experimental.pallas import tpu as pltpu
from jax.experimental.pallas import tpu_sc as plsc

B = 16384
NENT = 1000000
NREL = 100000
ENT_DIM = 64
REL_DIM = 96
H = 32
GAMMA = 12.0
NC, NS, L = 2, 16, 16
NW = NC * NS
BPW = B // NW
CH = 16
NCH = BPW // CH

TCOLS = NREL // 128
REM = NREL - TCOLS * 128
CPW = 25


def _rsqrt(x):
    i = plsc.bitcast(x, jnp.int32)
    i = 0x5F3759DF - (i >> 1)
    y = plsc.bitcast(i, jnp.float32)
    xh = 0.5 * x
    for _ in range(3):
        y = y * (1.5 - xh * y * y)
    return y


def _cst(d):
    return jnp.full((L,), d, jnp.int32)


def _remid_body(rel_t, rel_tail, out_hbm, big, tp, sem_i):
    wid = lax.axis_index("s") * NC + lax.axis_index("c")
    lane = lax.iota(jnp.int32, L)

    def fire(c, carry):
        tc = wid * CPW + c

        @pl.when(tc < TCOLS)
        def _():
            pltpu.async_copy(
                rel_t.at[pl.ds(H, H), pl.ds(tc * 128, 128)],
                big.at[c], sem_i)
        return carry

    lax.fori_loop(0, CPW, fire, 0)

    def drain(c, carry):
        tc = wid * CPW + c

        @pl.when(tc < TCOLS)
        def _():
            pltpu.make_async_copy(
                rel_t.at[pl.ds(H, H), pl.ds(0, 128)],
                big.at[c], sem_i).wait()
        return carry

    lax.fori_loop(0, CPW, drain, 0)

    def trans(c, carry):
        tc = wid * CPW + c

        @pl.when(tc < TCOLS)
        def _():
            for e0 in range(128 // L):
                rows = e0 * L + lane
                prow = rows >> 2
                pcol = (rows & 3) << 5
                for d in range(H):
                    plsc.store_scatter(
                        tp, [prow, pcol + d],
                        plsc.load_gather(big.at[c], [_cst(d), rows]))
            pltpu.sync_copy(tp, out_hbm.at[pl.ds(tc * 32, 32)])
        return carry

    lax.fori_loop(0, CPW, trans, 0)

    @pl.when(wid == 0)
    def _():
        pltpu.sync_copy(rel_tail, tp.at[pl.ds(0, 8)])
        pltpu.sync_copy(tp.at[pl.ds(0, 8)], out_hbm.at[pl.ds(TCOLS * 32, 8)])


def _score_body(heads, rels, tails, ent3, remid, out_hbm,
                hidx, ridx, tidx,
                hdiv0, rdiv0, tdiv0, hdiv1, rdiv1, tdiv1,
                hrows0, rrows0, trows0, hrows1, rrows1, trows1,
                outv, sem):
    wid = lax.axis_index("s") * NC + lax.axis_index("c")
    base = wid * BPW
    pltpu.sync_copy(heads.at[pl.ds(base, BPW)], hidx)
    pltpu.sync_copy(rels.at[pl.ds(base, BPW)], ridx)
    pltpu.sync_copy(tails.at[pl.ds(base, BPW)], tidx)

    lane = lax.iota(jnp.int32, L)
    zero = jnp.zeros((L,), jnp.float32)
    divs = [(hdiv0, rdiv0, tdiv0), (hdiv1, rdiv1, tdiv1)]
    rows_bufs = [(hrows0, rrows0, trows0), (hrows1, rrows1, trows1)]

    def build_and_fire(c):
        def go(p):
            hd, rd, td = divs[p]
            hr, rr, tr = rows_bufs[p]
            s = c * CH
            hv = hidx[pl.ds(s, CH)]
            tv = tidx[pl.ds(s, CH)]
            rd[...] = ridx[pl.ds(s, CH)]
            pltpu.async_copy(remid.at[rd], rr, sem)
            for j in range(CH):
                h8 = pl.multiple_of((hv[j] >> 3) << 3, 8)
                t8 = pl.multiple_of((tv[j] >> 3) << 3, 8)
                pltpu.async_copy(ent3.at[pl.ds(h8, 8), :], hr.at[j], sem)
                pltpu.async_copy(ent3.at[pl.ds(t8, 8), :], tr.at[j], sem)
        return go

    def drain(p):
        hd, rd, td = divs[p]
        hr, rr, tr = rows_bufs[p]
        pltpu.make_async_copy(remid.at[rd], rr, sem).wait()
        for j in range(CH):
            pltpu.make_async_copy(ent3.at[pl.ds(0, 8), :], hr.at[j], sem).wait()
            pltpu.make_async_copy(ent3.at[pl.ds(0, 8), :], tr.at[j], sem).wait()

    def compute(c, p):
        hr, rr, tr = rows_bufs[p]
        rows = lane
        s = c * CH
        hsub = hidx[pl.ds(s, CH)] & 7
        tsub = tidx[pl.ds(s, CH)] & 7

        def sumsq(ref, sub, lo):
            acc = zero
            for d in range(lo, lo + H):
                x = plsc.load_gather(ref, [rows, sub, _cst(d)])
                acc = acc + x * x
            return acc

        ra = _rsqrt(jnp.maximum(sumsq(hr, hsub, 0), 1e-12))
        rbh = _rsqrt(jnp.maximum(sumsq(hr, hsub, H), 1e-12))
        rat = _rsqrt(jnp.maximum(sumsq(tr, tsub, 0), 1e-12))
        rbt = _rsqrt(jnp.maximum(sumsq(tr, tsub, H), 1e-12))

        acc = zero
        for d in range(H):
            ah = plsc.load_gather(hr, [rows, hsub, _cst(d)])
            bh = plsc.load_gather(hr, [rows, hsub, _cst(H + d)])
            at = plsc.load_gather(tr, [rows, tsub, _cst(d)])
            bt = plsc.load_gather(tr, [rows, tsub, _cst(H + d)])
            m = plsc.load_gather(rr, [rows, _cst(d)])
            s_ = ((ah * ra) * (bt * rbt + 1.0)
                  - (at * rat) * (bh * rbh + 1.0) + m)
            acc = acc + s_ * s_
        norm = acc * _rsqrt(jnp.maximum(acc, 1e-30))
        outv[pl.ds(c * CH, CH)] = GAMMA - norm

    build_and_fire(0)(0)
    build_and_fire(1)(1)

    def pair_body(i, carry):
        c0 = 2 * i
        drain(0)
        compute(c0, 0)

        @pl.when(i < NCH // 2 - 1)
        def _():
            build_and_fire(c0 + 2)(0)
        drain(1)
        compute(c0 + 1, 1)

        @pl.when(i < NCH // 2 - 1)
        def _():
            build_and_fire(c0 + 3)(1)
        return carry

    lax.fori_loop(0, NCH // 2, pair_body, 0)
    pltpu.sync_copy(outv, out_hbm.at[pl.ds(base, BPW)])


@functools.partial(jax.jit, static_argnums=())
def kernel(sample, entity_embedding, relation_embedding):
    sample = sample.astype(jnp.int32)
    heads = sample[:, 0]
    rels = sample[:, 1]
    tails = sample[:, 2]

    mesh = plsc.VectorSubcoreMesh(
        core_axis_name="c", subcore_axis_name="s",
        num_cores=NC, num_subcores=NS)

    def _remid_tc_body(in_ref, o_ref):
        t = in_ref[...].T
        o_ref[...] = pltpu.repeat(t, 4, axis=1)

    remid = pl.pallas_call(
        _remid_tc_body,
        out_shape=jax.ShapeDtypeStruct((NREL, 128), jnp.float32),
        grid=(pl.cdiv(NREL, 512),),
        in_specs=[pl.BlockSpec((H, 512), lambda j: (1, j))],
        out_specs=pl.BlockSpec((512, 128), lambda j: (j, 0)),
    )(relation_embedding.T)

    ent3 = entity_embedding
    score = pl.kernel(
        _score_body,
        out_type=jax.ShapeDtypeStruct((B,), jnp.float32),
        mesh=mesh,
        scratch_types=[
            pltpu.VMEM((BPW,), jnp.int32),
            pltpu.VMEM((BPW,), jnp.int32),
            pltpu.VMEM((BPW,), jnp.int32),
            pltpu.VMEM((CH,), jnp.int32),
            pltpu.VMEM((CH,), jnp.int32),
            pltpu.VMEM((CH,), jnp.int32),
            pltpu.VMEM((CH,), jnp.int32),
            pltpu.VMEM((CH,), jnp.int32),
            pltpu.VMEM((CH,), jnp.int32),
            pltpu.VMEM((CH, 8, ENT_DIM), jnp.float32),
            pltpu.VMEM((CH, 128), jnp.float32),
            pltpu.VMEM((CH, 8, ENT_DIM), jnp.float32),
            pltpu.VMEM((CH, 8, ENT_DIM), jnp.float32),
            pltpu.VMEM((CH, 128), jnp.float32),
            pltpu.VMEM((CH, 8, ENT_DIM), jnp.float32),
            pltpu.VMEM((BPW,), jnp.float32),
            pltpu.SemaphoreType.DMA,
        ],
        compiler_params=pltpu.CompilerParams(
            needs_layout_passes=False, use_tc_tiling_on_sc=True),
    )(heads, rels, tails, ent3, remid)
    return score.reshape(B, 1)


# speedup vs baseline: 1.4712x; 1.4712x over previous
import functools

import jax
import jax.numpy as jnp
from jax import lax
from jax.experimental import pallas as pl
from jax.experimental.pallas import tpu as pltpu
from jax.experimental.pallas import tpu_sc as plsc

B = 16384
NENT = 1000000
NREL = 100000
ENT_DIM = 64
REL_DIM = 96
H = 32
GAMMA = 12.0
NC, NS, L = 2, 16, 16
NW = NC * NS
BPW = B // NW
CH = 16
NCH = BPW // CH

TCOLS = NREL // 128
REM = NREL - TCOLS * 128
CPW = 25


def _rsqrt(x):
    i = plsc.bitcast(x, jnp.int32)
    i = 0x5F3759DF - (i >> 1)
    y = plsc.bitcast(i, jnp.float32)
    xh = 0.5 * x
    for _ in range(3):
        y = y * (1.5 - xh * y * y)
    return y


def _cst(d):
    return jnp.full((L,), d, jnp.int32)


def _remid_body(rel_t, rel_tail, out_hbm, big, tp, sem_i):
    wid = lax.axis_index("s") * NC + lax.axis_index("c")
    lane = lax.iota(jnp.int32, L)

    def fire(c, carry):
        tc = wid * CPW + c

        @pl.when(tc < TCOLS)
        def _():
            pltpu.async_copy(
                rel_t.at[pl.ds(H, H), pl.ds(tc * 128, 128)],
                big.at[c], sem_i)
        return carry

    lax.fori_loop(0, CPW, fire, 0)

    def drain(c, carry):
        tc = wid * CPW + c

        @pl.when(tc < TCOLS)
        def _():
            pltpu.make_async_copy(
                rel_t.at[pl.ds(H, H), pl.ds(0, 128)],
                big.at[c], sem_i).wait()
        return carry

    lax.fori_loop(0, CPW, drain, 0)

    def trans(c, carry):
        tc = wid * CPW + c

        @pl.when(tc < TCOLS)
        def _():
            for e0 in range(128 // L):
                rows = e0 * L + lane
                prow = rows >> 2
                pcol = (rows & 3) << 5
                for d in range(H):
                    plsc.store_scatter(
                        tp, [prow, pcol + d],
                        plsc.load_gather(big.at[c], [_cst(d), rows]))
            pltpu.sync_copy(tp, out_hbm.at[pl.ds(tc * 32, 32)])
        return carry

    lax.fori_loop(0, CPW, trans, 0)

    @pl.when(wid == 0)
    def _():
        pltpu.sync_copy(rel_tail, tp.at[pl.ds(0, 8)])
        pltpu.sync_copy(tp.at[pl.ds(0, 8)], out_hbm.at[pl.ds(TCOLS * 32, 8)])


def _score_body(heads, rels, tails, ent3, remid, out_hbm,
                hidx, ridx, tidx,
                hdiv0, rdiv0, tdiv0, hdiv1, rdiv1, tdiv1,
                hrows0, rrows0, trows0, hrows1, rrows1, trows1,
                outv, sem):
    wid = lax.axis_index("s") * NC + lax.axis_index("c")
    base = wid * BPW
    pltpu.sync_copy(heads.at[pl.ds(base, BPW)], hidx)
    pltpu.sync_copy(rels.at[pl.ds(base, BPW)], ridx)
    pltpu.sync_copy(tails.at[pl.ds(base, BPW)], tidx)

    lane = lax.iota(jnp.int32, L)
    zero = jnp.zeros((L,), jnp.float32)
    divs = [(hdiv0, rdiv0, tdiv0), (hdiv1, rdiv1, tdiv1)]
    rows_bufs = [(hrows0, rrows0, trows0), (hrows1, rrows1, trows1)]

    def build_and_fire(c):
        def go(p):
            hd, rd, td = divs[p]
            hr, rr, tr = rows_bufs[p]
            s = c * CH
            hv = hidx[pl.ds(s, CH)]
            tv = tidx[pl.ds(s, CH)]
            rd[...] = ridx[pl.ds(s, CH)] >> 2
            pltpu.async_copy(remid.at[rd], rr, sem)
            for j in range(CH):
                pltpu.async_copy(ent3.at[hv[j] >> 3], hr.at[j], sem)
                pltpu.async_copy(ent3.at[tv[j] >> 3], tr.at[j], sem)
        return go

    def drain(p):
        hd, rd, td = divs[p]
        hr, rr, tr = rows_bufs[p]
        pltpu.make_async_copy(remid.at[rd], rr, sem).wait()
        for j in range(CH):
            pltpu.make_async_copy(ent3.at[0], hr.at[j], sem).wait()
            pltpu.make_async_copy(ent3.at[0], tr.at[j], sem).wait()

    def compute(c, p):
        hr, rr, tr = rows_bufs[p]
        rows = lane
        s = c * CH
        hsub = hidx[pl.ds(s, CH)] & 7
        tsub = tidx[pl.ds(s, CH)] & 7
        rcol = (ridx[pl.ds(s, CH)] & 3) << 5

        def sumsq(ref, sub, lo):
            acc = zero
            for d in range(lo, lo + H):
                x = plsc.load_gather(ref, [rows, sub, _cst(d)])
                acc = acc + x * x
            return acc

        ra = _rsqrt(jnp.maximum(sumsq(hr, hsub, 0), 1e-12))
        rbh = _rsqrt(jnp.maximum(sumsq(hr, hsub, H), 1e-12))
        rat = _rsqrt(jnp.maximum(sumsq(tr, tsub, 0), 1e-12))
        rbt = _rsqrt(jnp.maximum(sumsq(tr, tsub, H), 1e-12))

        acc = zero
        for d in range(H):
            ah = plsc.load_gather(hr, [rows, hsub, _cst(d)])
            bh = plsc.load_gather(hr, [rows, hsub, _cst(H + d)])
            at = plsc.load_gather(tr, [rows, tsub, _cst(d)])
            bt = plsc.load_gather(tr, [rows, tsub, _cst(H + d)])
            m = plsc.load_gather(rr, [rows, rcol + d])
            s_ = ((ah * ra) * (bt * rbt + 1.0)
                  - (at * rat) * (bh * rbh + 1.0) + m)
            acc = acc + s_ * s_
        norm = acc * _rsqrt(jnp.maximum(acc, 1e-30))
        outv[pl.ds(c * CH, CH)] = GAMMA - norm

    build_and_fire(0)(0)
    build_and_fire(1)(1)

    def pair_body(i, carry):
        c0 = 2 * i
        drain(0)
        compute(c0, 0)

        @pl.when(i < NCH // 2 - 1)
        def _():
            build_and_fire(c0 + 2)(0)
        drain(1)
        compute(c0 + 1, 1)

        @pl.when(i < NCH // 2 - 1)
        def _():
            build_and_fire(c0 + 3)(1)
        return carry

    lax.fori_loop(0, NCH // 2, pair_body, 0)
    pltpu.sync_copy(outv, out_hbm.at[pl.ds(base, BPW)])


@functools.partial(jax.jit, static_argnums=())
def kernel(sample, entity_embedding, relation_embedding):
    sample = sample.astype(jnp.int32)
    heads = sample[:, 0]
    rels = sample[:, 1]
    tails = sample[:, 2]

    mesh = plsc.VectorSubcoreMesh(
        core_axis_name="c", subcore_axis_name="s",
        num_cores=NC, num_subcores=NS)

    rel_tail = lax.slice(
        relation_embedding, (TCOLS * 128, H), (NREL, 2 * H)).reshape(8, 128)
    remid = pl.kernel(
        _remid_body,
        out_type=jax.ShapeDtypeStruct((NREL // 4, 128), jnp.float32),
        mesh=mesh,
        scratch_types=[
            pltpu.VMEM((CPW, H, 128), jnp.float32),
            pltpu.VMEM((32, 128), jnp.float32),
            pltpu.SemaphoreType.DMA,
        ],
        compiler_params=pltpu.CompilerParams(
            needs_layout_passes=False, use_tc_tiling_on_sc=True),
    )(relation_embedding.T, rel_tail)

    ent3 = entity_embedding.reshape(NENT // 8, 8, ENT_DIM)
    score = pl.kernel(
        _score_body,
        out_type=jax.ShapeDtypeStruct((B,), jnp.float32),
        mesh=mesh,
        scratch_types=[
            pltpu.VMEM((BPW,), jnp.int32),
            pltpu.VMEM((BPW,), jnp.int32),
            pltpu.VMEM((BPW,), jnp.int32),
            pltpu.VMEM((CH,), jnp.int32),
            pltpu.VMEM((CH,), jnp.int32),
            pltpu.VMEM((CH,), jnp.int32),
            pltpu.VMEM((CH,), jnp.int32),
            pltpu.VMEM((CH,), jnp.int32),
            pltpu.VMEM((CH,), jnp.int32),
            pltpu.VMEM((CH, 8, ENT_DIM), jnp.float32),
            pltpu.VMEM((CH, 128), jnp.float32),
            pltpu.VMEM((CH, 8, ENT_DIM), jnp.float32),
            pltpu.VMEM((CH, 8, ENT_DIM), jnp.float32),
            pltpu.VMEM((CH, 128), jnp.float32),
            pltpu.VMEM((CH, 8, ENT_DIM), jnp.float32),
            pltpu.VMEM((BPW,), jnp.float32),
            pltpu.SemaphoreType.DMA,
        ],
        compiler_params=pltpu.CompilerParams(
            needs_layout_passes=False, use_tc_tiling_on_sc=True),
    )(heads, rels, tails, ent3, remid)
    return score.reshape(B, 1)


# speedup vs baseline: 1.8225x; 1.2388x over previous
import functools

import jax
import jax.numpy as jnp
from jax import lax
from jax.experimental import pallas as pl
from jax.experimental.pallas import tpu as pltpu
from jax.experimental.pallas import tpu_sc as plsc

B = 16384
NENT = 1000000
NREL = 100000
ENT_DIM = 64
REL_DIM = 96
H = 32
GAMMA = 12.0
NC, NS, L = 2, 16, 16
NW = NC * NS
BPW = B // NW
CH = 16
NCH = BPW // CH

TCOLS = NREL // 128
REM = NREL - TCOLS * 128
CPW = 25


def _rsqrt(x):
    i = plsc.bitcast(x, jnp.int32)
    i = 0x5F3759DF - (i >> 1)
    y = plsc.bitcast(i, jnp.float32)
    xh = 0.5 * x
    for _ in range(3):
        y = y * (1.5 - xh * y * y)
    return y


def _cst(d):
    return jnp.full((L,), d, jnp.int32)


def _remid_body(rel_t, rel_tail, out_hbm, big, tp, sem_i):
    wid = lax.axis_index("s") * NC + lax.axis_index("c")
    lane = lax.iota(jnp.int32, L)

    def fire(c, carry):
        tc = wid * CPW + c

        @pl.when(tc < TCOLS)
        def _():
            pltpu.async_copy(
                rel_t.at[pl.ds(H, H), pl.ds(tc * 128, 128)],
                big.at[c], sem_i)
        return carry

    lax.fori_loop(0, CPW, fire, 0)

    def drain(c, carry):
        tc = wid * CPW + c

        @pl.when(tc < TCOLS)
        def _():
            pltpu.make_async_copy(
                rel_t.at[pl.ds(H, H), pl.ds(0, 128)],
                big.at[c], sem_i).wait()
        return carry

    lax.fori_loop(0, CPW, drain, 0)

    def trans(c, carry):
        tc = wid * CPW + c

        @pl.when(tc < TCOLS)
        def _():
            for e0 in range(128 // L):
                rows = e0 * L + lane
                prow = rows >> 2
                pcol = (rows & 3) << 5
                for d in range(H):
                    plsc.store_scatter(
                        tp, [prow, pcol + d],
                        plsc.load_gather(big.at[c], [_cst(d), rows]))
            pltpu.sync_copy(tp, out_hbm.at[pl.ds(tc * 32, 32)])
        return carry

    lax.fori_loop(0, CPW, trans, 0)

    @pl.when(wid == 0)
    def _():
        pltpu.sync_copy(rel_tail, tp.at[pl.ds(0, 8)])
        pltpu.sync_copy(tp.at[pl.ds(0, 8)], out_hbm.at[pl.ds(TCOLS * 32, 8)])


def _score_body(heads, rels, tails, ent3, remid, out_hbm,
                hidx, ridx, tidx,
                hdiv0, rdiv0, tdiv0, hdiv1, rdiv1, tdiv1,
                hrows0, rrows0, trows0, hrows1, rrows1, trows1,
                outv, sem):
    wid = lax.axis_index("s") * NC + lax.axis_index("c")
    base = wid * BPW
    pltpu.sync_copy(heads.at[pl.ds(base, BPW)], hidx)
    pltpu.sync_copy(rels.at[pl.ds(base, BPW)], ridx)
    pltpu.sync_copy(tails.at[pl.ds(base, BPW)], tidx)

    lane = lax.iota(jnp.int32, L)
    zero = jnp.zeros((L,), jnp.float32)
    divs = [(hdiv0, rdiv0, tdiv0), (hdiv1, rdiv1, tdiv1)]
    rows_bufs = [(hrows0, rrows0, trows0), (hrows1, rrows1, trows1)]

    def build_and_fire(c):
        def go(p):
            hd, rd, td = divs[p]
            hr, rr, tr = rows_bufs[p]
            s = c * CH
            hv = hidx[pl.ds(s, CH)]
            tv = tidx[pl.ds(s, CH)]
            rd[...] = ridx[pl.ds(s, CH)]
            pltpu.async_copy(remid.at[rd], rr, sem)
            for j in range(CH):
                pltpu.async_copy(ent3.at[hv[j] >> 3], hr.at[j], sem)
                pltpu.async_copy(ent3.at[tv[j] >> 3], tr.at[j], sem)
        return go

    def drain(p):
        hd, rd, td = divs[p]
        hr, rr, tr = rows_bufs[p]
        pltpu.make_async_copy(remid.at[rd], rr, sem).wait()
        for j in range(CH):
            pltpu.make_async_copy(ent3.at[0], hr.at[j], sem).wait()
            pltpu.make_async_copy(ent3.at[0], tr.at[j], sem).wait()

    def compute(c, p):
        hr, rr, tr = rows_bufs[p]
        rows = lane
        s = c * CH
        hsub = hidx[pl.ds(s, CH)] & 7
        tsub = tidx[pl.ds(s, CH)] & 7

        def sumsq(ref, sub, lo):
            acc = zero
            for d in range(lo, lo + H):
                x = plsc.load_gather(ref, [rows, sub, _cst(d)])
                acc = acc + x * x
            return acc

        ra = _rsqrt(jnp.maximum(sumsq(hr, hsub, 0), 1e-12))
        rbh = _rsqrt(jnp.maximum(sumsq(hr, hsub, H), 1e-12))
        rat = _rsqrt(jnp.maximum(sumsq(tr, tsub, 0), 1e-12))
        rbt = _rsqrt(jnp.maximum(sumsq(tr, tsub, H), 1e-12))

        acc = zero
        for d in range(H):
            ah = plsc.load_gather(hr, [rows, hsub, _cst(d)])
            bh = plsc.load_gather(hr, [rows, hsub, _cst(H + d)])
            at = plsc.load_gather(tr, [rows, tsub, _cst(d)])
            bt = plsc.load_gather(tr, [rows, tsub, _cst(H + d)])
            m = plsc.load_gather(rr, [rows, _cst(d)])
            s_ = ((ah * ra) * (bt * rbt + 1.0)
                  - (at * rat) * (bh * rbh + 1.0) + m)
            acc = acc + s_ * s_
        norm = acc * _rsqrt(jnp.maximum(acc, 1e-30))
        outv[pl.ds(c * CH, CH)] = GAMMA - norm

    build_and_fire(0)(0)
    build_and_fire(1)(1)

    def pair_body(i, carry):
        c0 = 2 * i
        drain(0)
        compute(c0, 0)

        @pl.when(i < NCH // 2 - 1)
        def _():
            build_and_fire(c0 + 2)(0)
        drain(1)
        compute(c0 + 1, 1)

        @pl.when(i < NCH // 2 - 1)
        def _():
            build_and_fire(c0 + 3)(1)
        return carry

    lax.fori_loop(0, NCH // 2, pair_body, 0)
    pltpu.sync_copy(outv, out_hbm.at[pl.ds(base, BPW)])


@functools.partial(jax.jit, static_argnums=())
def kernel(sample, entity_embedding, relation_embedding):
    sample = sample.astype(jnp.int32)
    heads = sample[:, 0]
    rels = sample[:, 1]
    tails = sample[:, 2]

    mesh = plsc.VectorSubcoreMesh(
        core_axis_name="c", subcore_axis_name="s",
        num_cores=NC, num_subcores=NS)

    def _remid_tc_body(in_ref, o_ref):
        t = in_ref[...].T
        o_ref[...] = pltpu.repeat(t, 4, axis=1)

    remid = pl.pallas_call(
        _remid_tc_body,
        out_shape=jax.ShapeDtypeStruct((NREL, 128), jnp.float32),
        grid=(pl.cdiv(NREL, 512),),
        in_specs=[pl.BlockSpec((H, 512), lambda j: (1, j))],
        out_specs=pl.BlockSpec((512, 128), lambda j: (j, 0)),
    )(relation_embedding.T)

    ent3 = entity_embedding.reshape(NENT // 8, 8, ENT_DIM)
    score = pl.kernel(
        _score_body,
        out_type=jax.ShapeDtypeStruct((B,), jnp.float32),
        mesh=mesh,
        scratch_types=[
            pltpu.VMEM((BPW,), jnp.int32),
            pltpu.VMEM((BPW,), jnp.int32),
            pltpu.VMEM((BPW,), jnp.int32),
            pltpu.VMEM((CH,), jnp.int32),
            pltpu.VMEM((CH,), jnp.int32),
            pltpu.VMEM((CH,), jnp.int32),
            pltpu.VMEM((CH,), jnp.int32),
            pltpu.VMEM((CH,), jnp.int32),
            pltpu.VMEM((CH,), jnp.int32),
            pltpu.VMEM((CH, 8, ENT_DIM), jnp.float32),
            pltpu.VMEM((CH, 128), jnp.float32),
            pltpu.VMEM((CH, 8, ENT_DIM), jnp.float32),
            pltpu.VMEM((CH, 8, ENT_DIM), jnp.float32),
            pltpu.VMEM((CH, 128), jnp.float32),
            pltpu.VMEM((CH, 8, ENT_DIM), jnp.float32),
            pltpu.VMEM((BPW,), jnp.float32),
            pltpu.SemaphoreType.DMA,
        ],
        compiler_params=pltpu.CompilerParams(
            needs_layout_passes=False, use_tc_tiling_on_sc=True),
    )(heads, rels, tails, ent3, remid)
    return score.reshape(B, 1)
